# Initial kernel scaffold; baseline (speedup 1.0000x reference)
import functools

import jax
import jax.numpy as jnp
from jax import lax
from jax.experimental import pallas as pl
from jax.experimental.pallas import tpu as pltpu
from jax.experimental.pallas import tpu_sc as plsc

_E = 8
_TOPK = 2
_EPS = 1e-5
_TILE = 256
_TN = 512

_NC = 2
_NS = 16
_NW = _NC * _NS


def _norm(x):
    m = jnp.mean(x, axis=-1, keepdims=True)
    xc = x - m
    v = jnp.mean(xc * xc, axis=-1, keepdims=True)
    return xc * lax.rsqrt(v + _EPS)


def _router_body(x_ref, ing_ref, inb_ref, rg_ref, rb_ref, rw_ref, rbias_ref,
                 z_ref, idx_ref, w_ref):
    x = x_ref[...]
    xn = _norm(x) * ing_ref[...] + inb_ref[...]
    z = _norm(xn)
    rn = z * rg_ref[...] + rb_ref[...]
    logits = jnp.dot(rn, rw_ref[...], preferred_element_type=jnp.float32)
    logits = logits + rbias_ref[...]
    n = logits.shape[0]
    col = lax.broadcasted_iota(jnp.int32, (n, _E), 1)
    v0 = jnp.max(logits, axis=-1, keepdims=True)
    i0 = jnp.min(jnp.where(logits == v0, col, _E), axis=-1, keepdims=True)
    l2 = jnp.where(col == i0, -jnp.inf, logits)
    v1 = jnp.max(l2, axis=-1, keepdims=True)
    i1 = jnp.min(jnp.where(l2 == v1, col, _E), axis=-1, keepdims=True)
    d = jnp.exp(v1 - v0)
    w0 = 1.0 / (1.0 + d)
    z_ref[...] = z.astype(jnp.bfloat16)
    idx_ref[...] = jnp.concatenate([i0, i1], axis=-1)
    w_ref[...] = jnp.concatenate([w0, 1.0 - w0], axis=-1)


def _router(x, in_g, in_b, r_g, r_b, r_W, r_bias):
    n, d = x.shape
    vec = lambda: pl.BlockSpec((1, d), lambda i: (0, 0))
    return pl.pallas_call(
        _router_body,
        grid=(n // _TN,),
        in_specs=[
            pl.BlockSpec((_TN, d), lambda i: (i, 0)),
            vec(), vec(), vec(), vec(),
            pl.BlockSpec((d, _E), lambda i: (0, 0)),
            pl.BlockSpec((1, _E), lambda i: (0, 0)),
        ],
        out_specs=[
            pl.BlockSpec((_TN, d), lambda i: (i, 0)),
            pl.BlockSpec((_TN, _TOPK), lambda i: (i, 0)),
            pl.BlockSpec((_TN, _TOPK), lambda i: (i, 0)),
        ],
        out_shape=[
            jax.ShapeDtypeStruct((n, d), jnp.bfloat16),
            jax.ShapeDtypeStruct((n, _TOPK), jnp.int32),
            jax.ShapeDtypeStruct((n, _TOPK), jnp.float32),
        ],
    )(x, in_g.reshape(1, d), in_b.reshape(1, d), r_g.reshape(1, d),
      r_b.reshape(1, d), r_W, r_bias.reshape(1, _E))


def _gelu(x):
    return 0.5 * x * (1.0 + lax.erf(x * 0.7071067811865476))


def _expert_body(te_ref, zs_ref, gate_ref,
                 g1_ref, b1_ref, w1_ref, bb1_ref,
                 g2_ref, b2_ref, w2_ref, bb2_ref,
                 g3_ref, b3_ref, w3_ref, bb3_ref,
                 out_ref):
    z = zs_ref[...].astype(jnp.float32)
    h = z * g1_ref[...] + b1_ref[...]
    h = jnp.dot(h.astype(jnp.bfloat16), w1_ref[0],
                preferred_element_type=jnp.float32) + bb1_ref[...]
    h = _gelu(h)
    h = _norm(h) * g2_ref[...] + b2_ref[...]
    h = jnp.dot(h.astype(jnp.bfloat16), w2_ref[0],
                preferred_element_type=jnp.float32) + bb2_ref[...]
    h = _gelu(h)
    h = _norm(h) * g3_ref[...] + b3_ref[...]
    y = jnp.dot(h, w3_ref[0], preferred_element_type=jnp.float32)
    out_ref[...] = (y + bb3_ref[0]) * gate_ref[...]


def _experts(tile_expert, zs, gate, e_ln1_g, e_ln1_b, w1_bf, e_b1,
             e_ln2_g, e_ln2_b, w2_bf, e_b2, e_ln3_g, e_ln3_b, e_W3, e_b3):
    p, d = zs.shape
    h = w1_bf.shape[-1]
    k = w2_bf.shape[-1]
    grid = p // _TILE
    ev = lambda m: pl.BlockSpec((1, m), lambda t, te: (te[t], 0))
    grid_spec = pltpu.PrefetchScalarGridSpec(
        num_scalar_prefetch=1,
        grid=(grid,),
        in_specs=[
            pl.BlockSpec((_TILE, d), lambda t, te: (t, 0)),
            pl.BlockSpec((_TILE, 1), lambda t, te: (t, 0)),
            ev(d), ev(d),
            pl.BlockSpec((1, d, h), lambda t, te: (te[t], 0, 0)),
            ev(h),
            ev(h), ev(h),
            pl.BlockSpec((1, h, k), lambda t, te: (te[t], 0, 0)),
            ev(k),
            ev(k), ev(k),
            pl.BlockSpec((1, k, 1), lambda t, te: (te[t], 0, 0)),
            pl.BlockSpec((1, 1, 1), lambda t, te: (te[t], 0, 0)),
        ],
        out_specs=pl.BlockSpec((_TILE, 1), lambda t, te: (t, 0)),
    )
    return pl.pallas_call(
        _expert_body,
        grid_spec=grid_spec,
        out_shape=jax.ShapeDtypeStruct((p, 1), jnp.float32),
    )(tile_expert, zs, gate, e_ln1_g, e_ln1_b, w1_bf, e_b1,
      e_ln2_g, e_ln2_b, w2_bf, e_b2, e_ln3_g, e_ln3_b, e_W3,
      e_b3.reshape(_E, 1, 1))


def _sc_gather(z_bf, src, p):
    n, d = z_bf.shape
    rows_per_w = p // _NW
    chunk = 96
    nchunks = rows_per_w // chunk
    mesh = plsc.VectorSubcoreMesh(core_axis_name="c", subcore_axis_name="s")

    @functools.partial(
        pl.kernel, mesh=mesh,
        out_type=jax.ShapeDtypeStruct((p, d), jnp.bfloat16),
        scratch_types=[
            pltpu.VMEM((chunk,), jnp.int32),
            pltpu.VMEM((chunk, d), jnp.bfloat16),
            pltpu.SemaphoreType.DMA,
        ],
    )
    def k(z_hbm, src_hbm, out_hbm, idx_v, rows_v, sem):
        wid = lax.axis_index("s") * _NC + lax.axis_index("c")
        for c in range(nchunks):
            base = wid * rows_per_w + c * chunk
            pltpu.sync_copy(src_hbm.at[pl.ds(base, chunk)], idx_v)
            pltpu.async_copy(z_hbm.at[idx_v], rows_v, sem).wait()
            pltpu.sync_copy(rows_v, out_hbm.at[pl.ds(base, chunk)])

    return k(z_bf, src)


def _sc_combine(yw, d0, d1):
    (p,) = yw.shape
    (n,) = d0.shape
    cn = n // _NW
    mesh = plsc.VectorSubcoreMesh(core_axis_name="c", subcore_axis_name="s")

    @functools.partial(
        pl.kernel, mesh=mesh,
        out_type=jax.ShapeDtypeStruct((n,), jnp.float32),
        scratch_types=[
            pltpu.VMEM((p,), jnp.float32),
            pltpu.VMEM((cn,), jnp.int32),
            pltpu.VMEM((cn,), jnp.int32),
            pltpu.VMEM((cn,), jnp.float32),
        ],
    )
    def k(yw_hbm, d0_hbm, d1_hbm, out_hbm, yw_v, d0_v, d1_v, out_v):
        wid = lax.axis_index("s") * _NC + lax.axis_index("c")
        base = wid * cn
        pltpu.sync_copy(yw_hbm, yw_v)
        pltpu.sync_copy(d0_hbm.at[pl.ds(base, cn)], d0_v)
        pltpu.sync_copy(d1_hbm.at[pl.ds(base, cn)], d1_v)
        for j in range(cn // 16):
            i0 = d0_v[pl.ds(j * 16, 16)]
            i1 = d1_v[pl.ds(j * 16, 16)]
            out_v[pl.ds(j * 16, 16)] = (
                plsc.load_gather(yw_v, [i0]) + plsc.load_gather(yw_v, [i1]))
        pltpu.sync_copy(out_v, out_hbm.at[pl.ds(base, cn)])

    return k(yw, d0, d1)


def kernel(x, in_g, in_b, r_g, r_b, r_W, r_bias,
           e_ln1_g, e_ln1_b, e_W1, e_b1,
           e_ln2_g, e_ln2_b, e_W2, e_b2,
           e_ln3_g, e_ln3_b, e_W3, e_b3):
    n, d = x.shape
    z_bf, idx, w = _router(x, in_g, in_b, r_g, r_b, r_W, r_bias)

    e_flat = idx.reshape(-1)
    oh = (e_flat[:, None] == jnp.arange(_E, dtype=jnp.int32)[None, :])
    oh = oh.astype(jnp.int32)
    csum = jnp.cumsum(oh, axis=0)
    rank = jnp.sum(csum * oh, axis=1) - 1
    counts = csum[-1]
    padded = ((counts + _TILE - 1) // _TILE) * _TILE
    ends = jnp.cumsum(padded)
    offs = ends - padded
    dest = jnp.sum(oh * offs[None, :], axis=1) + rank

    p = _TOPK * n + _E * _TILE
    pair_tok = (jnp.arange(_TOPK * n, dtype=jnp.int32) // _TOPK)
    src = jnp.zeros((p,), jnp.int32).at[dest].set(pair_tok)
    gate = jnp.zeros((p,), jnp.float32).at[dest].set(w.reshape(-1))

    g = p // _TILE
    tile_start = jnp.arange(g, dtype=jnp.int32) * _TILE
    tile_expert = jnp.sum(
        (tile_start[:, None] >= ends[None, :]).astype(jnp.int32), axis=1)
    tile_expert = jnp.minimum(tile_expert, _E - 1)

    zs = _sc_gather(z_bf, src, p)

    yw = _experts(tile_expert, zs, gate.reshape(p, 1),
                  e_ln1_g, e_ln1_b, e_W1.astype(jnp.bfloat16), e_b1,
                  e_ln2_g, e_ln2_b, e_W2.astype(jnp.bfloat16), e_b2,
                  e_ln3_g, e_ln3_b, e_W3, e_b3)

    dest2 = dest.reshape(n, _TOPK)
    out = _sc_combine(yw.reshape(p), dest2[:, 0], dest2[:, 1])
    return out.reshape(n, 1)


# speedup vs baseline: 2.3251x; 2.3251x over previous
import functools

import jax
import jax.numpy as jnp
from jax import lax
from jax.experimental import pallas as pl
from jax.experimental.pallas import tpu as pltpu
from jax.experimental.pallas import tpu_sc as plsc

_E = 8
_TOPK = 2
_EPS = 1e-5
_TILE = 256
_TN = 512

_NC = 2
_NS = 16
_NW = _NC * _NS


def _norm(x):
    m = jnp.mean(x, axis=-1, keepdims=True)
    xc = x - m
    v = jnp.mean(xc * xc, axis=-1, keepdims=True)
    return xc * lax.rsqrt(v + _EPS)


def _router_body(x_ref, ing_ref, inb_ref, rg_ref, rb_ref, rw_ref, rbias_ref,
                 z_ref, idx_ref, w_ref):
    x = x_ref[...]
    xn = _norm(x) * ing_ref[...] + inb_ref[...]
    z = _norm(xn)
    rn = z * rg_ref[...] + rb_ref[...]
    logits = jnp.dot(rn, rw_ref[...], preferred_element_type=jnp.float32)
    logits = logits + rbias_ref[...]
    n = logits.shape[0]
    col = lax.broadcasted_iota(jnp.int32, (n, _E), 1)
    v0 = jnp.max(logits, axis=-1, keepdims=True)
    i0 = jnp.min(jnp.where(logits == v0, col, _E), axis=-1, keepdims=True)
    l2 = jnp.where(col == i0, -jnp.inf, logits)
    v1 = jnp.max(l2, axis=-1, keepdims=True)
    i1 = jnp.min(jnp.where(l2 == v1, col, _E), axis=-1, keepdims=True)
    d = jnp.exp(v1 - v0)
    w0 = 1.0 / (1.0 + d)
    z_ref[...] = z
    idx_ref[...] = jnp.concatenate([i0, i1], axis=-1)
    w_ref[...] = jnp.concatenate([w0, 1.0 - w0], axis=-1)


def _router(x, in_g, in_b, r_g, r_b, r_W, r_bias):
    n, d = x.shape
    vec = lambda: pl.BlockSpec((1, d), lambda i: (0, 0))
    return pl.pallas_call(
        _router_body,
        grid=(n // _TN,),
        in_specs=[
            pl.BlockSpec((_TN, d), lambda i: (i, 0)),
            vec(), vec(), vec(), vec(),
            pl.BlockSpec((d, _E), lambda i: (0, 0)),
            pl.BlockSpec((1, _E), lambda i: (0, 0)),
        ],
        out_specs=[
            pl.BlockSpec((_TN, d), lambda i: (i, 0)),
            pl.BlockSpec((_TN, _TOPK), lambda i: (i, 0)),
            pl.BlockSpec((_TN, _TOPK), lambda i: (i, 0)),
        ],
        out_shape=[
            jax.ShapeDtypeStruct((n, d), jnp.float32),
            jax.ShapeDtypeStruct((n, _TOPK), jnp.int32),
            jax.ShapeDtypeStruct((n, _TOPK), jnp.float32),
        ],
    )(x, in_g.reshape(1, d), in_b.reshape(1, d), r_g.reshape(1, d),
      r_b.reshape(1, d), r_W, r_bias.reshape(1, _E))


def _gelu(x):
    return 0.5 * x * (1.0 + lax.erf(x * 0.7071067811865476))


def _expert_body(te_ref, zs_ref, gate_ref,
                 g1_ref, b1_ref, w1_ref, bb1_ref,
                 g2_ref, b2_ref, w2_ref, bb2_ref,
                 g3_ref, b3_ref, w3_ref, bb3_ref,
                 out_ref):
    z = zs_ref[...].astype(jnp.float32)
    h = z * g1_ref[0] + b1_ref[0]
    h = jnp.dot(h.astype(jnp.bfloat16), w1_ref[0],
                preferred_element_type=jnp.float32) + bb1_ref[0]
    h = _gelu(h)
    h = _norm(h) * g2_ref[0] + b2_ref[0]
    h = jnp.dot(h.astype(jnp.bfloat16), w2_ref[0],
                preferred_element_type=jnp.float32) + bb2_ref[0]
    h = _gelu(h)
    h = _norm(h) * g3_ref[0] + b3_ref[0]
    y = jnp.dot(h, w3_ref[0], preferred_element_type=jnp.float32)
    out_ref[...] = (y + bb3_ref[0]) * gate_ref[...]


def _experts(tile_expert, zs, gate, e_ln1_g, e_ln1_b, w1_bf, e_b1,
             e_ln2_g, e_ln2_b, w2_bf, e_b2, e_ln3_g, e_ln3_b, e_W3, e_b3):
    p, d = zs.shape
    h = w1_bf.shape[-1]
    k = w2_bf.shape[-1]
    grid = p // _TILE
    ev = lambda m: pl.BlockSpec((1, 1, m), lambda t, te: (te[t], 0, 0))
    grid_spec = pltpu.PrefetchScalarGridSpec(
        num_scalar_prefetch=1,
        grid=(grid,),
        in_specs=[
            pl.BlockSpec((_TILE, d), lambda t, te: (t, 0)),
            pl.BlockSpec((_TILE, 1), lambda t, te: (t, 0)),
            ev(d), ev(d),
            pl.BlockSpec((1, d, h), lambda t, te: (te[t], 0, 0)),
            ev(h),
            ev(h), ev(h),
            pl.BlockSpec((1, h, k), lambda t, te: (te[t], 0, 0)),
            ev(k),
            ev(k), ev(k),
            pl.BlockSpec((1, k, 1), lambda t, te: (te[t], 0, 0)),
            pl.BlockSpec((1, 1, 1), lambda t, te: (te[t], 0, 0)),
        ],
        out_specs=pl.BlockSpec((_TILE, 1), lambda t, te: (t, 0)),
    )
    return pl.pallas_call(
        _expert_body,
        grid_spec=grid_spec,
        out_shape=jax.ShapeDtypeStruct((p, 1), jnp.float32),
    )(tile_expert, zs, gate,
      e_ln1_g.reshape(_E, 1, d), e_ln1_b.reshape(_E, 1, d), w1_bf,
      e_b1.reshape(_E, 1, h),
      e_ln2_g.reshape(_E, 1, h), e_ln2_b.reshape(_E, 1, h), w2_bf,
      e_b2.reshape(_E, 1, k),
      e_ln3_g.reshape(_E, 1, k), e_ln3_b.reshape(_E, 1, k), e_W3,
      e_b3.reshape(_E, 1, 1))


def _sc_gather(z_bf, src, p):
    n, d = z_bf.shape
    rows_per_w = p // _NW
    chunk = 96
    nchunks = rows_per_w // chunk
    mesh = plsc.VectorSubcoreMesh(core_axis_name="c", subcore_axis_name="s")

    @functools.partial(
        pl.kernel, mesh=mesh,
        out_type=jax.ShapeDtypeStruct((p, d), jnp.float32),
        scratch_types=[
            pltpu.VMEM((chunk,), jnp.int32),
            pltpu.VMEM((chunk, d), jnp.float32),
            pltpu.SemaphoreType.DMA,
        ],
    )
    def k(z_hbm, src_hbm, out_hbm, idx_v, rows_v, sem):
        wid = lax.axis_index("s") * _NC + lax.axis_index("c")
        for c in range(nchunks):
            base = wid * rows_per_w + c * chunk
            pltpu.sync_copy(src_hbm.at[pl.ds(base, chunk)], idx_v)
            pltpu.async_copy(z_hbm.at[idx_v], rows_v, sem).wait()
            pltpu.sync_copy(rows_v, out_hbm.at[pl.ds(base, chunk)])

    return k(z_bf, src)


def _sc_combine(yw, d0, d1):
    (p,) = yw.shape
    (n,) = d0.shape
    cn = n // _NW
    mesh = plsc.VectorSubcoreMesh(core_axis_name="c", subcore_axis_name="s")

    chunk = 128
    nchunks = cn // chunk

    @functools.partial(
        pl.kernel, mesh=mesh,
        out_type=jax.ShapeDtypeStruct((n,), jnp.float32),
        scratch_types=[
            pltpu.VMEM((chunk,), jnp.int32),
            pltpu.VMEM((chunk,), jnp.int32),
            pltpu.VMEM((chunk,), jnp.float32),
            pltpu.VMEM((chunk,), jnp.float32),
            pltpu.VMEM((chunk,), jnp.float32),
            pltpu.SemaphoreType.DMA,
        ],
    )
    def k(yw_hbm, d0_hbm, d1_hbm, out_hbm, i0_v, i1_v, g0_v, g1_v, out_v,
          sem):
        wid = lax.axis_index("s") * _NC + lax.axis_index("c")
        for h in range(nchunks):
            base = wid * cn + h * chunk
            pltpu.sync_copy(d0_hbm.at[pl.ds(base, chunk)], i0_v)
            pltpu.sync_copy(d1_hbm.at[pl.ds(base, chunk)], i1_v)
            pltpu.async_copy(yw_hbm.at[i0_v], g0_v, sem).wait()
            pltpu.async_copy(yw_hbm.at[i1_v], g1_v, sem).wait()
            for j in range(chunk // 16):
                out_v[pl.ds(j * 16, 16)] = (
                    g0_v[pl.ds(j * 16, 16)] + g1_v[pl.ds(j * 16, 16)])
            pltpu.sync_copy(out_v, out_hbm.at[pl.ds(base, chunk)])

    return k(yw, d0, d1)


def kernel(x, in_g, in_b, r_g, r_b, r_W, r_bias,
           e_ln1_g, e_ln1_b, e_W1, e_b1,
           e_ln2_g, e_ln2_b, e_W2, e_b2,
           e_ln3_g, e_ln3_b, e_W3, e_b3):
    n, d = x.shape
    z_bf, idx, w = _router(x, in_g, in_b, r_g, r_b, r_W, r_bias)

    e_flat = idx.reshape(-1)
    oh = (e_flat[:, None] == jnp.arange(_E, dtype=jnp.int32)[None, :])
    oh = oh.astype(jnp.int32)
    csum = jnp.cumsum(oh, axis=0)
    rank = jnp.sum(csum * oh, axis=1) - 1
    counts = csum[-1]
    padded = ((counts + _TILE - 1) // _TILE) * _TILE
    ends = jnp.cumsum(padded)
    offs = ends - padded
    dest = jnp.sum(oh * offs[None, :], axis=1) + rank

    p = _TOPK * n + _E * _TILE
    pair_tok = (jnp.arange(_TOPK * n, dtype=jnp.int32) // _TOPK)
    src = jnp.zeros((p,), jnp.int32).at[dest].set(pair_tok)
    gate = jnp.zeros((p,), jnp.float32).at[dest].set(w.reshape(-1))

    g = p // _TILE
    tile_start = jnp.arange(g, dtype=jnp.int32) * _TILE
    tile_expert = jnp.sum(
        (tile_start[:, None] >= ends[None, :]).astype(jnp.int32), axis=1)
    tile_expert = jnp.minimum(tile_expert, _E - 1)

    zs = _sc_gather(z_bf, src, p)

    yw = _experts(tile_expert, zs, gate.reshape(p, 1),
                  e_ln1_g, e_ln1_b, e_W1.astype(jnp.bfloat16), e_b1,
                  e_ln2_g, e_ln2_b, e_W2.astype(jnp.bfloat16), e_b2,
                  e_ln3_g, e_ln3_b, e_W3, e_b3)

    dest2 = dest.reshape(n, _TOPK)
    out = _sc_combine(yw.reshape(p), dest2[:, 0], dest2[:, 1])
    return out.reshape(n, 1)


# speedup vs baseline: 2.6020x; 1.1191x over previous
import functools

import jax
import jax.numpy as jnp
from jax import lax
from jax.experimental import pallas as pl
from jax.experimental.pallas import tpu as pltpu
from jax.experimental.pallas import tpu_sc as plsc

_E = 8
_TOPK = 2
_EPS = 1e-5
_TILE = 256
_TN = 512

_NC = 2
_NS = 16
_NW = _NC * _NS


def _norm(x):
    m = jnp.mean(x, axis=-1, keepdims=True)
    xc = x - m
    v = jnp.mean(xc * xc, axis=-1, keepdims=True)
    return xc * lax.rsqrt(v + _EPS)


def _router_body(x_ref, ing_ref, inb_ref, rg_ref, rb_ref, rw_ref, rbias_ref,
                 z_ref, idx_ref, w_ref):
    x = x_ref[...]
    xn = _norm(x) * ing_ref[...] + inb_ref[...]
    z = _norm(xn)
    rn = z * rg_ref[...] + rb_ref[...]
    logits = jnp.dot(rn, rw_ref[...], preferred_element_type=jnp.float32)
    logits = logits + rbias_ref[...]
    n = logits.shape[0]
    col = lax.broadcasted_iota(jnp.int32, (n, _E), 1)
    v0 = jnp.max(logits, axis=-1, keepdims=True)
    i0 = jnp.min(jnp.where(logits == v0, col, _E), axis=-1, keepdims=True)
    l2 = jnp.where(col == i0, -jnp.inf, logits)
    v1 = jnp.max(l2, axis=-1, keepdims=True)
    i1 = jnp.min(jnp.where(l2 == v1, col, _E), axis=-1, keepdims=True)
    d = jnp.exp(v1 - v0)
    w0 = 1.0 / (1.0 + d)
    half = z.shape[-1] // 2
    lo = lax.bitcast_convert_type(z[:, :half].astype(jnp.bfloat16),
                                  jnp.uint16).astype(jnp.uint32)
    hi = lax.bitcast_convert_type(z[:, half:].astype(jnp.bfloat16),
                                  jnp.uint16).astype(jnp.uint32)
    packed = jnp.bitwise_or(lo, jnp.left_shift(hi, 16))
    z_ref[...] = lax.bitcast_convert_type(packed, jnp.int32)
    idx_ref[...] = jnp.concatenate([i0, i1], axis=-1)
    w_ref[...] = jnp.concatenate([w0, 1.0 - w0], axis=-1)


def _router(x, in_g, in_b, r_g, r_b, r_W, r_bias):
    n, d = x.shape
    vec = lambda: pl.BlockSpec((1, d), lambda i: (0, 0))
    return pl.pallas_call(
        _router_body,
        grid=(n // _TN,),
        in_specs=[
            pl.BlockSpec((_TN, d), lambda i: (i, 0)),
            vec(), vec(), vec(), vec(),
            pl.BlockSpec((d, _E), lambda i: (0, 0)),
            pl.BlockSpec((1, _E), lambda i: (0, 0)),
        ],
        out_specs=[
            pl.BlockSpec((_TN, d // 2), lambda i: (i, 0)),
            pl.BlockSpec((_TN, _TOPK), lambda i: (i, 0)),
            pl.BlockSpec((_TN, _TOPK), lambda i: (i, 0)),
        ],
        out_shape=[
            jax.ShapeDtypeStruct((n, d // 2), jnp.int32),
            jax.ShapeDtypeStruct((n, _TOPK), jnp.int32),
            jax.ShapeDtypeStruct((n, _TOPK), jnp.float32),
        ],
    )(x, in_g.reshape(1, d), in_b.reshape(1, d), r_g.reshape(1, d),
      r_b.reshape(1, d), r_W, r_bias.reshape(1, _E))


def _gelu(x):
    return 0.5 * x * (1.0 + lax.erf(x * 0.7071067811865476))


def _expert_body(te_ref, zs_ref, gate_ref,
                 g1_ref, b1_ref, w1_ref, bb1_ref,
                 g2_ref, b2_ref, w2_ref, bb2_ref,
                 g3_ref, b3_ref, w3_ref, bb3_ref,
                 out_ref):
    u = lax.bitcast_convert_type(zs_ref[...], jnp.uint32)
    lo = lax.bitcast_convert_type(jnp.left_shift(u, 16), jnp.float32)
    hi = lax.bitcast_convert_type(
        jnp.bitwise_and(u, jnp.uint32(0xFFFF0000)), jnp.float32)
    z = jnp.concatenate([lo, hi], axis=-1)
    h = z * g1_ref[0] + b1_ref[0]
    h = jnp.dot(h.astype(jnp.bfloat16), w1_ref[0],
                preferred_element_type=jnp.float32) + bb1_ref[0]
    h = _gelu(h)
    h = _norm(h) * g2_ref[0] + b2_ref[0]
    h = jnp.dot(h.astype(jnp.bfloat16), w2_ref[0],
                preferred_element_type=jnp.float32) + bb2_ref[0]
    h = _gelu(h)
    h = _norm(h) * g3_ref[0] + b3_ref[0]
    y = jnp.dot(h, w3_ref[0], preferred_element_type=jnp.float32)
    out_ref[...] = (y + bb3_ref[0]) * gate_ref[...]


def _experts(tile_expert, zs, gate, e_ln1_g, e_ln1_b, w1_bf, e_b1,
             e_ln2_g, e_ln2_b, w2_bf, e_b2, e_ln3_g, e_ln3_b, e_W3, e_b3):
    p, dp = zs.shape
    d = e_ln1_g.shape[-1]
    h = w1_bf.shape[-1]
    k = w2_bf.shape[-1]
    grid = p // _TILE
    ev = lambda m: pl.BlockSpec((1, 1, m), lambda t, te: (te[t], 0, 0))
    grid_spec = pltpu.PrefetchScalarGridSpec(
        num_scalar_prefetch=1,
        grid=(grid,),
        in_specs=[
            pl.BlockSpec((_TILE, dp), lambda t, te: (t, 0)),
            pl.BlockSpec((_TILE, 1), lambda t, te: (t, 0)),
            ev(d), ev(d),
            pl.BlockSpec((1, d, h), lambda t, te: (te[t], 0, 0)),
            ev(h),
            ev(h), ev(h),
            pl.BlockSpec((1, h, k), lambda t, te: (te[t], 0, 0)),
            ev(k),
            ev(k), ev(k),
            pl.BlockSpec((1, k, 1), lambda t, te: (te[t], 0, 0)),
            pl.BlockSpec((1, 1, 1), lambda t, te: (te[t], 0, 0)),
        ],
        out_specs=pl.BlockSpec((_TILE, 1), lambda t, te: (t, 0)),
    )
    return pl.pallas_call(
        _expert_body,
        grid_spec=grid_spec,
        out_shape=jax.ShapeDtypeStruct((p, 1), jnp.float32),
    )(tile_expert, zs, gate,
      e_ln1_g.reshape(_E, 1, d), e_ln1_b.reshape(_E, 1, d), w1_bf,
      e_b1.reshape(_E, 1, h),
      e_ln2_g.reshape(_E, 1, h), e_ln2_b.reshape(_E, 1, h), w2_bf,
      e_b2.reshape(_E, 1, k),
      e_ln3_g.reshape(_E, 1, k), e_ln3_b.reshape(_E, 1, k), e_W3,
      e_b3.reshape(_E, 1, 1))


def _sc_gather(z_packed, src, p):
    n, dp = z_packed.shape
    rows_per_w = p // _NW
    chunk = 96
    nchunks = rows_per_w // chunk
    mesh = plsc.VectorSubcoreMesh(core_axis_name="c", subcore_axis_name="s")

    @functools.partial(
        pl.kernel, mesh=mesh,
        out_type=jax.ShapeDtypeStruct((p, dp), jnp.int32),
        scratch_types=[
            pltpu.VMEM((nchunks, chunk), jnp.int32),
            pltpu.VMEM((chunk, dp), jnp.int32),
            pltpu.VMEM((chunk, dp), jnp.int32),
            pltpu.SemaphoreType.DMA,
            pltpu.SemaphoreType.DMA,
            pltpu.SemaphoreType.DMA,
            pltpu.SemaphoreType.DMA,
        ],
    )
    def k(z_hbm, src_hbm, out_hbm, idx_v, buf0, buf1, sg0, sg1, sw0, sw1):
        wid = lax.axis_index("s") * _NC + lax.axis_index("c")
        base_w = wid * rows_per_w
        for c in range(nchunks):
            pltpu.sync_copy(src_hbm.at[pl.ds(base_w + c * chunk, chunk)],
                            idx_v.at[c])
        bufs = (buf0, buf1)
        gsems = (sg0, sg1)
        wsems = (sw0, sw1)
        gd = [None] * nchunks
        wd = [None] * nchunks
        gd[0] = pltpu.async_copy(z_hbm.at[idx_v.at[0]], bufs[0], gsems[0])
        for c in range(nchunks):
            b = c & 1
            gd[c].wait()
            if c + 1 < nchunks:
                b2 = (c + 1) & 1
                if c >= 1:
                    wd[c - 1].wait()
                gd[c + 1] = pltpu.async_copy(
                    z_hbm.at[idx_v.at[c + 1]], bufs[b2], gsems[b2])
            wd[c] = pltpu.async_copy(
                bufs[b], out_hbm.at[pl.ds(base_w + c * chunk, chunk)],
                wsems[b])
        wd[nchunks - 2].wait()
        wd[nchunks - 1].wait()

    return k(z_packed, src)


def _sc_combine(yw, d0, d1):
    (p,) = yw.shape
    (n,) = d0.shape
    cn = n // _NW
    mesh = plsc.VectorSubcoreMesh(core_axis_name="c", subcore_axis_name="s")

    chunk = 128
    nchunks = cn // chunk

    @functools.partial(
        pl.kernel, mesh=mesh,
        out_type=jax.ShapeDtypeStruct((n,), jnp.float32),
        scratch_types=[
            pltpu.VMEM((chunk,), jnp.int32),
            pltpu.VMEM((chunk,), jnp.int32),
            pltpu.VMEM((chunk,), jnp.float32),
            pltpu.VMEM((chunk,), jnp.float32),
            pltpu.VMEM((chunk,), jnp.float32),
            pltpu.SemaphoreType.DMA,
        ],
    )
    def k(yw_hbm, d0_hbm, d1_hbm, out_hbm, i0_v, i1_v, g0_v, g1_v, out_v,
          sem):
        wid = lax.axis_index("s") * _NC + lax.axis_index("c")
        for h in range(nchunks):
            base = wid * cn + h * chunk
            pltpu.sync_copy(d0_hbm.at[pl.ds(base, chunk)], i0_v)
            pltpu.sync_copy(d1_hbm.at[pl.ds(base, chunk)], i1_v)
            pltpu.async_copy(yw_hbm.at[i0_v], g0_v, sem).wait()
            pltpu.async_copy(yw_hbm.at[i1_v], g1_v, sem).wait()
            for j in range(chunk // 16):
                out_v[pl.ds(j * 16, 16)] = (
                    g0_v[pl.ds(j * 16, 16)] + g1_v[pl.ds(j * 16, 16)])
            pltpu.sync_copy(out_v, out_hbm.at[pl.ds(base, chunk)])

    return k(yw, d0, d1)


def kernel(x, in_g, in_b, r_g, r_b, r_W, r_bias,
           e_ln1_g, e_ln1_b, e_W1, e_b1,
           e_ln2_g, e_ln2_b, e_W2, e_b2,
           e_ln3_g, e_ln3_b, e_W3, e_b3):
    n, d = x.shape
    z_bf, idx, w = _router(x, in_g, in_b, r_g, r_b, r_W, r_bias)

    e_flat = idx.reshape(-1)
    oh = (e_flat[:, None] == jnp.arange(_E, dtype=jnp.int32)[None, :])
    oh = oh.astype(jnp.int32)
    csum = jnp.cumsum(oh, axis=0)
    rank = jnp.sum(csum * oh, axis=1) - 1
    counts = csum[-1]
    padded = ((counts + _TILE - 1) // _TILE) * _TILE
    ends = jnp.cumsum(padded)
    offs = ends - padded
    dest = jnp.sum(oh * offs[None, :], axis=1) + rank

    p = _TOPK * n + _E * _TILE
    pair_tok = (jnp.arange(_TOPK * n, dtype=jnp.int32) // _TOPK)
    src = jnp.zeros((p,), jnp.int32).at[dest].set(pair_tok)
    gate = jnp.zeros((p,), jnp.float32).at[dest].set(w.reshape(-1))

    g = p // _TILE
    tile_start = jnp.arange(g, dtype=jnp.int32) * _TILE
    tile_expert = jnp.sum(
        (tile_start[:, None] >= ends[None, :]).astype(jnp.int32), axis=1)
    tile_expert = jnp.minimum(tile_expert, _E - 1)

    zs = _sc_gather(z_bf, src, p)

    yw = _experts(tile_expert, zs, gate.reshape(p, 1),
                  e_ln1_g, e_ln1_b, e_W1.astype(jnp.bfloat16), e_b1,
                  e_ln2_g, e_ln2_b, e_W2.astype(jnp.bfloat16), e_b2,
                  e_ln3_g, e_ln3_b, e_W3, e_b3)

    dest2 = dest.reshape(n, _TOPK)
    out = _sc_combine(yw.reshape(p), dest2[:, 0], dest2[:, 1])
    return out.reshape(n, 1)


# speedup vs baseline: 4.5614x; 1.7531x over previous
import functools

import jax
import jax.numpy as jnp
from jax import lax
from jax.experimental import pallas as pl
from jax.experimental.pallas import tpu as pltpu
from jax.experimental.pallas import tpu_sc as plsc

_E = 8
_TOPK = 2
_EPS = 1e-5
_TILE = 256
_TN = 512

_NC = 2
_NS = 16
_NW = _NC * _NS


def _norm(x):
    m = jnp.mean(x, axis=-1, keepdims=True)
    xc = x - m
    v = jnp.mean(xc * xc, axis=-1, keepdims=True)
    return xc * lax.rsqrt(v + _EPS)


def _router_body(x_ref, ing_ref, inb_ref, rg_ref, rb_ref, rw_ref, rbias_ref,
                 z_ref, idx_ref, w_ref):
    x = x_ref[...]
    xn = _norm(x) * ing_ref[...] + inb_ref[...]
    z = _norm(xn)
    rn = z * rg_ref[...] + rb_ref[...]
    logits = jnp.dot(rn, rw_ref[...], preferred_element_type=jnp.float32)
    logits = logits + rbias_ref[...]
    n = logits.shape[0]
    col = lax.broadcasted_iota(jnp.int32, (n, _E), 1)
    v0 = jnp.max(logits, axis=-1, keepdims=True)
    i0 = jnp.min(jnp.where(logits == v0, col, _E), axis=-1, keepdims=True)
    l2 = jnp.where(col == i0, -jnp.inf, logits)
    v1 = jnp.max(l2, axis=-1, keepdims=True)
    i1 = jnp.min(jnp.where(l2 == v1, col, _E), axis=-1, keepdims=True)
    d = jnp.exp(v1 - v0)
    w0 = 1.0 / (1.0 + d)
    half = z.shape[-1] // 2
    lo = lax.bitcast_convert_type(z[:, :half].astype(jnp.bfloat16),
                                  jnp.uint16).astype(jnp.uint32)
    hi = lax.bitcast_convert_type(z[:, half:].astype(jnp.bfloat16),
                                  jnp.uint16).astype(jnp.uint32)
    packed = jnp.bitwise_or(lo, jnp.left_shift(hi, 16))
    z_ref[...] = lax.bitcast_convert_type(packed, jnp.int32)
    idx_ref[...] = jnp.concatenate([i0, i1], axis=-1)
    w_ref[...] = jnp.concatenate([w0, 1.0 - w0], axis=-1)


def _router(x, in_g, in_b, r_g, r_b, r_W, r_bias):
    n, d = x.shape
    vec = lambda: pl.BlockSpec((1, d), lambda i: (0, 0))
    return pl.pallas_call(
        _router_body,
        grid=(n // _TN,),
        in_specs=[
            pl.BlockSpec((_TN, d), lambda i: (i, 0)),
            vec(), vec(), vec(), vec(),
            pl.BlockSpec((d, _E), lambda i: (0, 0)),
            pl.BlockSpec((1, _E), lambda i: (0, 0)),
        ],
        out_specs=[
            pl.BlockSpec((_TN, d // 2), lambda i: (i, 0)),
            pl.BlockSpec((_TN, _TOPK), lambda i: (i, 0)),
            pl.BlockSpec((_TN, _TOPK), lambda i: (i, 0)),
        ],
        out_shape=[
            jax.ShapeDtypeStruct((n, d // 2), jnp.int32),
            jax.ShapeDtypeStruct((n, _TOPK), jnp.int32),
            jax.ShapeDtypeStruct((n, _TOPK), jnp.float32),
        ],
    )(x, in_g.reshape(1, d), in_b.reshape(1, d), r_g.reshape(1, d),
      r_b.reshape(1, d), r_W, r_bias.reshape(1, _E))


def _gelu(x):
    return 0.5 * x * (1.0 + lax.erf(x * 0.7071067811865476))


def _expert_body(te_ref, zs_ref,
                 g1_ref, b1_ref, w1_ref, bb1_ref,
                 g2_ref, b2_ref, w2_ref, bb2_ref,
                 g3_ref, b3_ref, w3_ref, bb3_ref,
                 out_ref):
    u = lax.bitcast_convert_type(zs_ref[...], jnp.uint32)
    lo = lax.bitcast_convert_type(jnp.left_shift(u, 16), jnp.float32)
    hi = lax.bitcast_convert_type(
        jnp.bitwise_and(u, jnp.uint32(0xFFFF0000)), jnp.float32)
    z = jnp.concatenate([lo, hi], axis=-1)
    h = z * g1_ref[0] + b1_ref[0]
    h = jnp.dot(h.astype(jnp.bfloat16), w1_ref[0],
                preferred_element_type=jnp.float32) + bb1_ref[0]
    h = _gelu(h)
    h = _norm(h) * g2_ref[0] + b2_ref[0]
    h = jnp.dot(h.astype(jnp.bfloat16), w2_ref[0],
                preferred_element_type=jnp.float32) + bb2_ref[0]
    h = _gelu(h)
    h = _norm(h) * g3_ref[0] + b3_ref[0]
    y = jnp.dot(h, w3_ref[0], preferred_element_type=jnp.float32)
    out_ref[...] = y + bb3_ref[0]


def _experts(tile_expert, zs, e_ln1_g, e_ln1_b, w1_bf, e_b1,
             e_ln2_g, e_ln2_b, w2_bf, e_b2, e_ln3_g, e_ln3_b, e_W3, e_b3):
    p, dp = zs.shape
    d = e_ln1_g.shape[-1]
    h = w1_bf.shape[-1]
    k = w2_bf.shape[-1]
    grid = p // _TILE
    ev = lambda m: pl.BlockSpec((1, 1, m), lambda t, te: (te[t], 0, 0))
    grid_spec = pltpu.PrefetchScalarGridSpec(
        num_scalar_prefetch=1,
        grid=(grid,),
        in_specs=[
            pl.BlockSpec((_TILE, dp), lambda t, te: (t, 0)),
            ev(d), ev(d),
            pl.BlockSpec((1, d, h), lambda t, te: (te[t], 0, 0)),
            ev(h),
            ev(h), ev(h),
            pl.BlockSpec((1, h, k), lambda t, te: (te[t], 0, 0)),
            ev(k),
            ev(k), ev(k),
            pl.BlockSpec((1, k, 1), lambda t, te: (te[t], 0, 0)),
            pl.BlockSpec((1, 1, 1), lambda t, te: (te[t], 0, 0)),
        ],
        out_specs=pl.BlockSpec((_TILE, 1), lambda t, te: (t, 0)),
    )
    return pl.pallas_call(
        _expert_body,
        grid_spec=grid_spec,
        out_shape=jax.ShapeDtypeStruct((p, 1), jnp.float32),
    )(tile_expert, zs,
      e_ln1_g.reshape(_E, 1, d), e_ln1_b.reshape(_E, 1, d), w1_bf,
      e_b1.reshape(_E, 1, h),
      e_ln2_g.reshape(_E, 1, h), e_ln2_b.reshape(_E, 1, h), w2_bf,
      e_b2.reshape(_E, 1, k),
      e_ln3_g.reshape(_E, 1, k), e_ln3_b.reshape(_E, 1, k), e_W3,
      e_b3.reshape(_E, 1, 1))


def _sc_scatter(z_packed, d0, d1, p):
    n, dp = z_packed.shape
    tw = n // _NW
    chunk = 128
    nchunks = tw // chunk
    mesh = plsc.VectorSubcoreMesh(core_axis_name="c", subcore_axis_name="s")

    @functools.partial(
        pl.kernel, mesh=mesh,
        out_type=jax.ShapeDtypeStruct((p, dp), jnp.int32),
        scratch_types=[
            pltpu.VMEM((tw, dp), jnp.int32),
            pltpu.VMEM((nchunks, chunk), jnp.int32),
            pltpu.VMEM((nchunks, chunk), jnp.int32),
            pltpu.SemaphoreType.DMA,
            pltpu.SemaphoreType.DMA,
        ],
    )
    def k(z_hbm, d0_hbm, d1_hbm, out_hbm, rows_v, i0_v, i1_v, sr, sw):
        wid = lax.axis_index("s") * _NC + lax.axis_index("c")
        base_t = wid * tw
        rd = pltpu.async_copy(z_hbm.at[pl.ds(base_t, tw)], rows_v, sr)
        for c in range(nchunks):
            pltpu.sync_copy(d0_hbm.at[pl.ds(base_t + c * chunk, chunk)],
                            i0_v.at[c])
            pltpu.sync_copy(d1_hbm.at[pl.ds(base_t + c * chunk, chunk)],
                            i1_v.at[c])
        rd.wait()
        wd = []
        for c in range(nchunks):
            src_slice = rows_v.at[pl.ds(c * chunk, chunk)]
            wd.append(pltpu.async_copy(src_slice, out_hbm.at[i0_v.at[c]], sw))
            wd.append(pltpu.async_copy(src_slice, out_hbm.at[i1_v.at[c]], sw))
        for w in wd:
            w.wait()

    return k(z_packed, d0, d1)


def _sc_combine(y, d0, d1, w0, w1):
    (p,) = y.shape
    (n,) = d0.shape
    cn = n // _NW
    mesh = plsc.VectorSubcoreMesh(core_axis_name="c", subcore_axis_name="s")

    chunk = 128
    nchunks = cn // chunk

    @functools.partial(
        pl.kernel, mesh=mesh,
        out_type=jax.ShapeDtypeStruct((n,), jnp.float32),
        scratch_types=[
            pltpu.VMEM((chunk,), jnp.int32),
            pltpu.VMEM((chunk,), jnp.int32),
            pltpu.VMEM((chunk,), jnp.float32),
            pltpu.VMEM((chunk,), jnp.float32),
            pltpu.VMEM((chunk,), jnp.float32),
            pltpu.VMEM((chunk,), jnp.float32),
            pltpu.VMEM((chunk,), jnp.float32),
            pltpu.SemaphoreType.DMA,
        ],
    )
    def k(y_hbm, d0_hbm, d1_hbm, w0_hbm, w1_hbm, out_hbm,
          i0_v, i1_v, g0_v, g1_v, w0_v, w1_v, out_v, sem):
        wid = lax.axis_index("s") * _NC + lax.axis_index("c")
        for h in range(nchunks):
            base = wid * cn + h * chunk
            pltpu.sync_copy(d0_hbm.at[pl.ds(base, chunk)], i0_v)
            pltpu.sync_copy(d1_hbm.at[pl.ds(base, chunk)], i1_v)
            pltpu.sync_copy(w0_hbm.at[pl.ds(base, chunk)], w0_v)
            pltpu.sync_copy(w1_hbm.at[pl.ds(base, chunk)], w1_v)
            pltpu.async_copy(y_hbm.at[i0_v], g0_v, sem).wait()
            pltpu.async_copy(y_hbm.at[i1_v], g1_v, sem).wait()
            for j in range(chunk // 16):
                s = pl.ds(j * 16, 16)
                out_v[s] = w0_v[s] * g0_v[s] + w1_v[s] * g1_v[s]
            pltpu.sync_copy(out_v, out_hbm.at[pl.ds(base, chunk)])

    return k(y, d0, d1, w0, w1)


def kernel(x, in_g, in_b, r_g, r_b, r_W, r_bias,
           e_ln1_g, e_ln1_b, e_W1, e_b1,
           e_ln2_g, e_ln2_b, e_W2, e_b2,
           e_ln3_g, e_ln3_b, e_W3, e_b3):
    n, d = x.shape
    z_bf, idx, w = _router(x, in_g, in_b, r_g, r_b, r_W, r_bias)

    e_flat = idx.reshape(-1)
    oh = (e_flat[:, None] == jnp.arange(_E, dtype=jnp.int32)[None, :])
    oh = oh.astype(jnp.int32)
    csum = jnp.cumsum(oh, axis=0)
    rank = jnp.sum(csum * oh, axis=1) - 1
    counts = csum[-1]
    padded = ((counts + _TILE - 1) // _TILE) * _TILE
    ends = jnp.cumsum(padded)
    offs = ends - padded
    dest = jnp.sum(oh * offs[None, :], axis=1) + rank

    p = _TOPK * n + _E * _TILE
    g = p // _TILE
    tile_start = jnp.arange(g, dtype=jnp.int32) * _TILE
    tile_expert = jnp.sum(
        (tile_start[:, None] >= ends[None, :]).astype(jnp.int32), axis=1)
    tile_expert = jnp.minimum(tile_expert, _E - 1)

    dest2 = dest.reshape(n, _TOPK)
    d0 = dest2[:, 0]
    d1 = dest2[:, 1]

    zs = _sc_scatter(z_bf, d0, d1, p)

    y = _experts(tile_expert, zs,
                 e_ln1_g, e_ln1_b, e_W1.astype(jnp.bfloat16), e_b1,
                 e_ln2_g, e_ln2_b, e_W2.astype(jnp.bfloat16), e_b2,
                 e_ln3_g, e_ln3_b, e_W3, e_b3)

    out = _sc_combine(y.reshape(p), d0, d1, w[:, 0], w[:, 1])
    return out.reshape(n, 1)
